# baseline (speedup 1.0000x reference)
import functools

import jax
import jax.numpy as jnp
from jax import lax
from jax.experimental import pallas as pl
from jax.experimental.pallas import tpu as pltpu
from jax.experimental.pallas import tpu_sc as plsc

N = 10000
E = 320000
DF = 128
DE = 16
H1 = 50
H1P = 64
EH = 16
H2 = 80
NLAYER = 3

NC, NS = 2, 16
NW = NC * NS
CH = 80
CPW = E // NW // CH
RPT = N // NS

_SELU_ALPHA = 1.6732632423543772
_SELU_SCALE = 1.0507009873554805


def _selu(v):
    return _SELU_SCALE * jnp.where(v > 0.0, v, _SELU_ALPHA * (jnp.exp(v) - 1.0))


def _sc_mesh():
    return plsc.VectorSubcoreMesh(
        core_axis_name="c", subcore_axis_name="s", num_cores=NC, num_subcores=NS
    )


def _make_seg_gather(width):
    @functools.partial(
        pl.kernel,
        out_type=jax.ShapeDtypeStruct((NC, N, width), jnp.float32),
        mesh=_sc_mesh(),
        compiler_params=pltpu.CompilerParams(use_tc_tiling_on_sc=False),
        scratch_types=[
            pltpu.VMEM((CPW, CH), jnp.int32),
            pltpu.VMEM((CPW, CH), jnp.int32),
            pltpu.VMEM((3, CH, width), jnp.float32),
            pltpu.VMEM_SHARED((N, width), jnp.float32),
            pltpu.SemaphoreType.DMA,
            pltpu.SemaphoreType.DMA,
            pltpu.SemaphoreType.DMA,
        ],
    )
    def seg_gather(rows_hbm, src_hbm, dst_hbm, z_hbm, out_hbm,
                   src_v, dst_v, buf_v, acc_sh, sem0, sem1, sem2):
        sems = (sem0, sem1, sem2)
        c = lax.axis_index("c")
        s = lax.axis_index("s")
        wid = c * NS + s
        pltpu.sync_copy(z_hbm.at[pl.ds(s * RPT, RPT)],
                        acc_sh.at[pl.ds(s * RPT, RPT)])
        pltpu.sync_copy(src_hbm.at[pl.ds(wid * CPW, CPW)], src_v)
        pltpu.sync_copy(dst_hbm.at[pl.ds(wid * CPW, CPW)], dst_v)
        plsc.subcore_barrier()

        def gather(j, b):
            pltpu.async_copy(rows_hbm.at[src_v.at[j]], buf_v.at[b], sems[b])

        def drain_and_scatter(j, b):
            pltpu.make_async_copy(rows_hbm.at[src_v.at[j]], buf_v.at[b],
                                  sems[b]).wait()
            pltpu.sync_copy(buf_v.at[b], acc_sh.at[dst_v.at[j]], add=True)

        gather(0, 0)
        gather(1, 1)

        def body(g, carry):
            for b in range(3):
                j = g * 3 + b
                gather(j + 2, (b + 2) % 3)
                drain_and_scatter(j, b)
            return carry

        lax.fori_loop(0, (CPW - 2) // 3, body, 0)
        drain_and_scatter(CPW - 2, (CPW - 2) % 3)
        drain_and_scatter(CPW - 1, (CPW - 1) % 3)
        plsc.subcore_barrier()
        pltpu.sync_copy(acc_sh.at[pl.ds(s * RPT, RPT)],
                        out_hbm.at[c, pl.ds(s * RPT, RPT)])

    return seg_gather


def _make_seg_layer0():
    @functools.partial(
        pl.kernel,
        out_type=[
            jax.ShapeDtypeStruct((NC, N, H1P), jnp.float32),
            jax.ShapeDtypeStruct((NC, N, EH), jnp.float32),
            jax.ShapeDtypeStruct((NC, N, EH), jnp.float32),
        ],
        mesh=_sc_mesh(),
        compiler_params=pltpu.CompilerParams(use_tc_tiling_on_sc=False),
        scratch_types=[
            pltpu.VMEM((CPW, CH), jnp.int32),
            pltpu.VMEM((CPW, CH), jnp.int32),
            pltpu.VMEM((3, CH, H1P), jnp.float32),
            pltpu.VMEM((3, CH, EH), jnp.float32),
            pltpu.VMEM((CH, EH), jnp.float32),
            pltpu.VMEM_SHARED((N, H1P), jnp.float32),
            pltpu.VMEM_SHARED((N, EH), jnp.float32),
            pltpu.VMEM_SHARED((N, EH), jnp.float32),
            pltpu.SemaphoreType.DMA,
            pltpu.SemaphoreType.DMA,
            pltpu.SemaphoreType.DMA,
            pltpu.SemaphoreType.DMA,
            pltpu.SemaphoreType.DMA,
            pltpu.SemaphoreType.DMA,
        ],
    )
    def seg_layer0(rows_hbm, ea_hbm, src_hbm, dst_hbm, zw_hbm, ze_hbm,
                   ones_hbm, outw_hbm, oute_hbm, outd_hbm,
                   src_v, dst_v, buf_v, eb_v, ones_v, accw_sh, acce_sh,
                   accd_sh, gs0, gs1, gs2, ls0, ls1, ls2):
        gsems = (gs0, gs1, gs2)
        lsems = (ls0, ls1, ls2)
        c = lax.axis_index("c")
        s = lax.axis_index("s")
        wid = c * NS + s
        pltpu.sync_copy(zw_hbm.at[pl.ds(s * RPT, RPT)],
                        accw_sh.at[pl.ds(s * RPT, RPT)])
        pltpu.sync_copy(ze_hbm.at[pl.ds(s * RPT, RPT)],
                        acce_sh.at[pl.ds(s * RPT, RPT)])
        pltpu.sync_copy(ze_hbm.at[pl.ds(s * RPT, RPT)],
                        accd_sh.at[pl.ds(s * RPT, RPT)])
        pltpu.sync_copy(src_hbm.at[pl.ds(wid * CPW, CPW)], src_v)
        pltpu.sync_copy(dst_hbm.at[pl.ds(wid * CPW, CPW)], dst_v)
        pltpu.sync_copy(ones_hbm, ones_v)
        plsc.subcore_barrier()

        def fetch(j, b):
            pltpu.async_copy(rows_hbm.at[src_v.at[j]], buf_v.at[b], gsems[b])
            pltpu.async_copy(ea_hbm.at[pl.ds((wid * CPW + j) * CH, CH)],
                             eb_v.at[b], lsems[b])

        def drain_and_scatter(j, b):
            pltpu.make_async_copy(rows_hbm.at[src_v.at[j]], buf_v.at[b],
                                  gsems[b]).wait()
            pltpu.make_async_copy(ea_hbm.at[pl.ds((wid * CPW + j) * CH, CH)],
                                  eb_v.at[b], lsems[b]).wait()
            pltpu.sync_copy(buf_v.at[b], accw_sh.at[dst_v.at[j]], add=True)
            pltpu.sync_copy(eb_v.at[b], acce_sh.at[dst_v.at[j]], add=True)
            pltpu.sync_copy(ones_v, accd_sh.at[dst_v.at[j]], add=True)

        fetch(0, 0)
        fetch(1, 1)

        def body(g, carry):
            for b in range(3):
                j = g * 3 + b
                fetch(j + 2, (b + 2) % 3)
                drain_and_scatter(j, b)
            return carry

        lax.fori_loop(0, (CPW - 2) // 3, body, 0)
        drain_and_scatter(CPW - 2, (CPW - 2) % 3)
        drain_and_scatter(CPW - 1, (CPW - 1) % 3)
        plsc.subcore_barrier()
        pltpu.sync_copy(accw_sh.at[pl.ds(s * RPT, RPT)],
                        outw_hbm.at[c, pl.ds(s * RPT, RPT)])
        pltpu.sync_copy(acce_sh.at[pl.ds(s * RPT, RPT)],
                        oute_hbm.at[c, pl.ds(s * RPT, RPT)])
        pltpu.sync_copy(accd_sh.at[pl.ds(s * RPT, RPT)],
                        outd_hbm.at[c, pl.ds(s * RPT, RPT)])

    return seg_layer0


_BN = 2000
_GRID = N // _BN

_full = lambda shape: pl.BlockSpec(shape, lambda i: tuple(0 for _ in shape))


def _hv0_body(x_ref, vw_ref, vb_ref, out_ref):
    out_ref[...] = (
        jnp.dot(x_ref[...], vw_ref[...], preferred_element_type=jnp.float32, precision=lax.Precision.HIGHEST)
        + vb_ref[...]
    )


def _layer_h(h_ref, aw_ref, se_ref, dg_ref, ew_ref, eb_ref,
             ua_ref, ub_ref, uc_ref, bias_ref):
    aw = aw_ref[0] + aw_ref[1]
    ae = (
        jnp.dot(se_ref[0] + se_ref[1], ew_ref[...],
                preferred_element_type=jnp.float32, precision=lax.Precision.HIGHEST)
        + (dg_ref[0] + dg_ref[1])[:, :1] * eb_ref[...]
    )
    z = (
        jnp.dot(h_ref[...], ua_ref[...], preferred_element_type=jnp.float32, precision=lax.Precision.HIGHEST)
        + jnp.dot(aw, ub_ref[...], preferred_element_type=jnp.float32, precision=lax.Precision.HIGHEST)
        + jnp.dot(ae, uc_ref[...], preferred_element_type=jnp.float32, precision=lax.Precision.HIGHEST)
        + bias_ref[...]
    )
    return _selu(z)


def _layer_body(h_ref, aw_ref, se_ref, dg_ref, ew_ref, eb_ref,
                ua_ref, ub_ref, uc_ref, bias_ref,
                vw_ref, vb_ref, h_out, hv_out):
    hn = _layer_h(h_ref, aw_ref, se_ref, dg_ref, ew_ref, eb_ref,
                  ua_ref, ub_ref, uc_ref, bias_ref)
    h_out[...] = hn
    hv_out[...] = (
        jnp.dot(hn, vw_ref[...], preferred_element_type=jnp.float32, precision=lax.Precision.HIGHEST)
        + vb_ref[...]
    )


def _final_body(h_ref, x_ref, aw_ref, se_ref, dg_ref, ew_ref, eb_ref,
                ua_ref, ub_ref, uc_ref,
                bias_ref, rh_ref, rx_ref, rb_ref, l0w_ref, l0b_ref,
                acc_ref, out_ref):
    i = pl.program_id(0)
    hn = _layer_h(h_ref, aw_ref, se_ref, dg_ref, ew_ref, eb_ref,
                  ua_ref, ub_ref, uc_ref, bias_ref)
    r = _selu(
        jnp.dot(hn, rh_ref[...], preferred_element_type=jnp.float32, precision=lax.Precision.HIGHEST)
        + jnp.dot(x_ref[...], rx_ref[...], preferred_element_type=jnp.float32, precision=lax.Precision.HIGHEST)
        + rb_ref[...]
    )
    part = jnp.sum(r, axis=0, keepdims=True)

    @pl.when(i == 0)
    def _():
        acc_ref[...] = part

    @pl.when(i > 0)
    def _():
        acc_ref[...] += part

    t = jnp.tanh(acc_ref[...])
    out_ref[...] = (
        jnp.dot(t, l0w_ref[...], preferred_element_type=jnp.float32, precision=lax.Precision.HIGHEST)
        + l0b_ref[...]
    )


def kernel(x, edge_index, edge_attr, params):
    f32 = jnp.float32
    src = edge_index[0].astype(jnp.int32).reshape(NW * CPW, CH)
    dst = edge_index[1].astype(jnp.int32).reshape(NW * CPW, CH)

    vw = jnp.pad(params["V_w"], ((0, 0), (0, 0), (0, H1P - H1)))
    vb = jnp.pad(params["V_b"], ((0, 0), (0, H1P - H1)))[:, None, :]
    ua = params["U_w"][:, :DF, :]
    ub = jnp.pad(params["U_w"][:, DF:DF + H1, :],
                 ((0, 0), (0, H1P - H1), (0, 0)))
    uc = params["U_w"][:, DF + H1:, :]
    u_bias = params["U_b"][:, None, :]
    ew = params["E_w"]
    eb = params["E_b"][None, :]
    rw = jnp.pad(params["R_w"], ((0, 0), (0, DF - H2)))
    rh, rx = rw[:DF], rw[DF:]
    rb = jnp.pad(params["R_b"], (0, DF - H2))[None, :]
    l0w = jnp.pad(params["lin0_w"], ((0, DF - H2), (0, 0)))
    l0w = jnp.pad(l0w, ((0, 0), (0, DF - 1)))
    l0b = jnp.pad(params["lin0_b"], (0, DF - 1))[None, :]

    zw = jnp.zeros((N, H1P), f32)
    ze = jnp.zeros((N, EH), f32)
    ones16 = jnp.ones((CH, EH), f32)

    hv = pl.pallas_call(
        _hv0_body,
        grid=(_GRID,),
        in_specs=[
            pl.BlockSpec((_BN, DF), lambda i: (i, 0)),
            _full((DF, H1P)),
            _full((1, H1P)),
        ],
        out_specs=pl.BlockSpec((_BN, H1P), lambda i: (i, 0)),
        out_shape=jax.ShapeDtypeStruct((N, H1P), f32),
    )(x, vw[0], vb[0])

    aggw, se, dg = _make_seg_layer0()(hv, edge_attr, src, dst, zw, ze, ones16)
    seg_gather = _make_seg_gather(H1P)

    layer_call = pl.pallas_call(
        _layer_body,
        grid=(_GRID,),
        in_specs=[
            pl.BlockSpec((_BN, DF), lambda i: (i, 0)),
            pl.BlockSpec((NC, _BN, H1P), lambda i: (0, i, 0)),
            pl.BlockSpec((NC, _BN, EH), lambda i: (0, i, 0)),
            pl.BlockSpec((NC, _BN, EH), lambda i: (0, i, 0)),
            _full((EH, EH)),
            _full((1, EH)),
            _full((DF, DF)),
            _full((H1P, DF)),
            _full((EH, DF)),
            _full((1, DF)),
            _full((DF, H1P)),
            _full((1, H1P)),
        ],
        out_specs=[
            pl.BlockSpec((_BN, DF), lambda i: (i, 0)),
            pl.BlockSpec((_BN, H1P), lambda i: (i, 0)),
        ],
        out_shape=[
            jax.ShapeDtypeStruct((N, DF), f32),
            jax.ShapeDtypeStruct((N, H1P), f32),
        ],
    )

    h = x
    for k in range(NLAYER - 1):
        if k > 0:
            aggw = seg_gather(hv, src, dst, zw)
        h, hv = layer_call(h, aggw, se, dg, ew, eb,
                           ua[k], ub[k], uc[k], u_bias[k],
                           vw[k + 1], vb[k + 1])

    aggw = seg_gather(hv, src, dst, zw)

    _, out = pl.pallas_call(
        _final_body,
        grid=(_GRID,),
        in_specs=[
            pl.BlockSpec((_BN, DF), lambda i: (i, 0)),
            pl.BlockSpec((_BN, DF), lambda i: (i, 0)),
            pl.BlockSpec((NC, _BN, H1P), lambda i: (0, i, 0)),
            pl.BlockSpec((NC, _BN, EH), lambda i: (0, i, 0)),
            pl.BlockSpec((NC, _BN, EH), lambda i: (0, i, 0)),
            _full((EH, EH)),
            _full((1, EH)),
            _full((DF, DF)),
            _full((H1P, DF)),
            _full((EH, DF)),
            _full((1, DF)),
            _full((DF, DF)),
            _full((DF, DF)),
            _full((1, DF)),
            _full((DF, DF)),
            _full((1, DF)),
        ],
        out_specs=[
            pl.BlockSpec((1, DF), lambda i: (0, 0)),
            pl.BlockSpec((1, DF), lambda i: (0, 0)),
        ],
        out_shape=[
            jax.ShapeDtypeStruct((1, DF), f32),
            jax.ShapeDtypeStruct((1, DF), f32),
        ],
    )(h, x, aggw, se, dg, ew, eb, ua[2], ub[2], uc[2], u_bias[2],
      rh, rx, rb, l0w, l0b)

    return out[:, :1]


# speedup vs baseline: 1.0166x; 1.0166x over previous
import functools

import jax
import jax.numpy as jnp
from jax import lax
from jax.experimental import pallas as pl
from jax.experimental.pallas import tpu as pltpu
from jax.experimental.pallas import tpu_sc as plsc

N = 10000
E = 320000
DF = 128
DE = 16
H1 = 50
H1P = 64
EH = 16
H2 = 80
NLAYER = 3

NC, NS = 2, 16
NW = NC * NS
EPW = E // NW
CH = 80
CPW = EPW // CH
RPT = N // NS

_SELU_ALPHA = 1.6732632423543772
_SELU_SCALE = 1.0507009873554805


def _selu(v):
    return _SELU_SCALE * jnp.where(v > 0.0, v, _SELU_ALPHA * (jnp.exp(v) - 1.0))


def _sc_mesh():
    return plsc.VectorSubcoreMesh(
        core_axis_name="c", subcore_axis_name="s", num_cores=NC, num_subcores=NS
    )


def _make_seg_gather(width):
    @functools.partial(
        pl.kernel,
        out_type=jax.ShapeDtypeStruct((NC, N, width), jnp.float32),
        mesh=_sc_mesh(),
        compiler_params=pltpu.CompilerParams(use_tc_tiling_on_sc=False),
        scratch_types=[
            pltpu.VMEM((EPW,), jnp.int32),
            pltpu.VMEM((EPW,), jnp.int32),
            pltpu.VMEM((3, CH, width), jnp.float32),
            pltpu.VMEM_SHARED((N, width), jnp.float32),
            pltpu.SemaphoreType.DMA,
            pltpu.SemaphoreType.DMA,
            pltpu.SemaphoreType.DMA,
        ],
    )
    def seg_gather(rows_hbm, ei_hbm, z_hbm, out_hbm,
                   src_v, dst_v, buf_v, acc_sh, sem0, sem1, sem2):
        sems = (sem0, sem1, sem2)
        c = lax.axis_index("c")
        s = lax.axis_index("s")
        wid = c * NS + s
        pltpu.sync_copy(z_hbm.at[pl.ds(s * RPT, RPT)],
                        acc_sh.at[pl.ds(s * RPT, RPT)])
        pltpu.sync_copy(ei_hbm.at[0, pl.ds(wid * EPW, EPW)], src_v)
        pltpu.sync_copy(ei_hbm.at[1, pl.ds(wid * EPW, EPW)], dst_v)
        plsc.subcore_barrier()

        def gather(j, b):
            pltpu.async_copy(rows_hbm.at[src_v.at[pl.ds(j * CH, CH)]],
                             buf_v.at[b], sems[b])

        def drain_and_scatter(j, b):
            pltpu.make_async_copy(rows_hbm.at[src_v.at[pl.ds(j * CH, CH)]],
                                  buf_v.at[b], sems[b]).wait()
            pltpu.sync_copy(buf_v.at[b],
                            acc_sh.at[dst_v.at[pl.ds(j * CH, CH)]], add=True)

        gather(0, 0)
        gather(1, 1)

        def body(g, carry):
            for b in range(3):
                j = g * 3 + b
                gather(j + 2, (b + 2) % 3)
                drain_and_scatter(j, b)
            return carry

        lax.fori_loop(0, (CPW - 2) // 3, body, 0)
        drain_and_scatter(CPW - 2, (CPW - 2) % 3)
        drain_and_scatter(CPW - 1, (CPW - 1) % 3)
        plsc.subcore_barrier()
        pltpu.sync_copy(acc_sh.at[pl.ds(s * RPT, RPT)],
                        out_hbm.at[c, pl.ds(s * RPT, RPT)])

    return seg_gather


def _make_seg_layer0():
    @functools.partial(
        pl.kernel,
        out_type=[
            jax.ShapeDtypeStruct((NC, N, H1P), jnp.float32),
            jax.ShapeDtypeStruct((NC, N, EH), jnp.float32),
            jax.ShapeDtypeStruct((NC, N, EH), jnp.float32),
        ],
        mesh=_sc_mesh(),
        compiler_params=pltpu.CompilerParams(use_tc_tiling_on_sc=False),
        scratch_types=[
            pltpu.VMEM((EPW,), jnp.int32),
            pltpu.VMEM((EPW,), jnp.int32),
            pltpu.VMEM((3, CH, H1P), jnp.float32),
            pltpu.VMEM((3, CH, EH), jnp.float32),
            pltpu.VMEM((CH, EH), jnp.float32),
            pltpu.VMEM_SHARED((N, H1P), jnp.float32),
            pltpu.VMEM_SHARED((N, EH), jnp.float32),
            pltpu.VMEM_SHARED((N, EH), jnp.float32),
            pltpu.SemaphoreType.DMA,
            pltpu.SemaphoreType.DMA,
            pltpu.SemaphoreType.DMA,
            pltpu.SemaphoreType.DMA,
            pltpu.SemaphoreType.DMA,
            pltpu.SemaphoreType.DMA,
        ],
    )
    def seg_layer0(rows_hbm, ea_hbm, ei_hbm, zw_hbm, ze_hbm,
                   ones_hbm, outw_hbm, oute_hbm, outd_hbm,
                   src_v, dst_v, buf_v, eb_v, ones_v, accw_sh, acce_sh,
                   accd_sh, gs0, gs1, gs2, ls0, ls1, ls2):
        gsems = (gs0, gs1, gs2)
        lsems = (ls0, ls1, ls2)
        c = lax.axis_index("c")
        s = lax.axis_index("s")
        wid = c * NS + s
        pltpu.sync_copy(zw_hbm.at[pl.ds(s * RPT, RPT)],
                        accw_sh.at[pl.ds(s * RPT, RPT)])
        pltpu.sync_copy(ze_hbm.at[pl.ds(s * RPT, RPT)],
                        acce_sh.at[pl.ds(s * RPT, RPT)])
        pltpu.sync_copy(ze_hbm.at[pl.ds(s * RPT, RPT)],
                        accd_sh.at[pl.ds(s * RPT, RPT)])
        pltpu.sync_copy(ei_hbm.at[0, pl.ds(wid * EPW, EPW)], src_v)
        pltpu.sync_copy(ei_hbm.at[1, pl.ds(wid * EPW, EPW)], dst_v)
        pltpu.sync_copy(ones_hbm, ones_v)
        plsc.subcore_barrier()

        def fetch(j, b):
            pltpu.async_copy(rows_hbm.at[src_v.at[pl.ds(j * CH, CH)]],
                             buf_v.at[b], gsems[b])
            pltpu.async_copy(ea_hbm.at[pl.ds(wid * EPW + j * CH, CH)],
                             eb_v.at[b], lsems[b])

        def drain_and_scatter(j, b):
            dchunk = dst_v.at[pl.ds(j * CH, CH)]
            pltpu.make_async_copy(rows_hbm.at[src_v.at[pl.ds(j * CH, CH)]],
                                  buf_v.at[b], gsems[b]).wait()
            pltpu.make_async_copy(ea_hbm.at[pl.ds(wid * EPW + j * CH, CH)],
                                  eb_v.at[b], lsems[b]).wait()
            pltpu.sync_copy(buf_v.at[b], accw_sh.at[dchunk], add=True)
            pltpu.sync_copy(eb_v.at[b], acce_sh.at[dchunk], add=True)
            pltpu.sync_copy(ones_v, accd_sh.at[dchunk], add=True)

        fetch(0, 0)
        fetch(1, 1)

        def body(g, carry):
            for b in range(3):
                j = g * 3 + b
                fetch(j + 2, (b + 2) % 3)
                drain_and_scatter(j, b)
            return carry

        lax.fori_loop(0, (CPW - 2) // 3, body, 0)
        drain_and_scatter(CPW - 2, (CPW - 2) % 3)
        drain_and_scatter(CPW - 1, (CPW - 1) % 3)
        plsc.subcore_barrier()
        pltpu.sync_copy(accw_sh.at[pl.ds(s * RPT, RPT)],
                        outw_hbm.at[c, pl.ds(s * RPT, RPT)])
        pltpu.sync_copy(acce_sh.at[pl.ds(s * RPT, RPT)],
                        oute_hbm.at[c, pl.ds(s * RPT, RPT)])
        pltpu.sync_copy(accd_sh.at[pl.ds(s * RPT, RPT)],
                        outd_hbm.at[c, pl.ds(s * RPT, RPT)])

    return seg_layer0


_BN = 2000
_GRID = N // _BN

_full = lambda shape: pl.BlockSpec(shape, lambda i: tuple(0 for _ in shape))


def _hv0_body(x_ref, vw_ref, vb_ref, out_ref):
    out_ref[...] = (
        jnp.dot(x_ref[...], vw_ref[...], preferred_element_type=jnp.float32, precision=lax.Precision.HIGHEST)
        + vb_ref[...]
    )


def _layer_h(h_ref, aw_ref, se_ref, dg_ref, ew_ref, eb_ref,
             ua_ref, ub_ref, uc_ref, bias_ref):
    aw = aw_ref[0] + aw_ref[1]
    ae = (
        jnp.dot(se_ref[0] + se_ref[1], ew_ref[...],
                preferred_element_type=jnp.float32, precision=lax.Precision.HIGHEST)
        + (dg_ref[0] + dg_ref[1])[:, :1] * eb_ref[...]
    )
    z = (
        jnp.dot(h_ref[...], ua_ref[...], preferred_element_type=jnp.float32, precision=lax.Precision.HIGHEST)
        + jnp.dot(aw, ub_ref[...], preferred_element_type=jnp.float32, precision=lax.Precision.HIGHEST)
        + jnp.dot(ae, uc_ref[...], preferred_element_type=jnp.float32, precision=lax.Precision.HIGHEST)
        + bias_ref[...]
    )
    return _selu(z)


def _layer_body(h_ref, aw_ref, se_ref, dg_ref, ew_ref, eb_ref,
                ua_ref, ub_ref, uc_ref, bias_ref,
                vw_ref, vb_ref, h_out, hv_out):
    hn = _layer_h(h_ref, aw_ref, se_ref, dg_ref, ew_ref, eb_ref,
                  ua_ref, ub_ref, uc_ref, bias_ref)
    h_out[...] = hn
    hv_out[...] = (
        jnp.dot(hn, vw_ref[...], preferred_element_type=jnp.float32, precision=lax.Precision.HIGHEST)
        + vb_ref[...]
    )


def _final_body(h_ref, x_ref, aw_ref, se_ref, dg_ref, ew_ref, eb_ref,
                ua_ref, ub_ref, uc_ref,
                bias_ref, rh_ref, rx_ref, rb_ref, l0w_ref, l0b_ref,
                acc_ref, out_ref):
    i = pl.program_id(0)
    hn = _layer_h(h_ref, aw_ref, se_ref, dg_ref, ew_ref, eb_ref,
                  ua_ref, ub_ref, uc_ref, bias_ref)
    r = _selu(
        jnp.dot(hn, rh_ref[...], preferred_element_type=jnp.float32, precision=lax.Precision.HIGHEST)
        + jnp.dot(x_ref[...], rx_ref[...], preferred_element_type=jnp.float32, precision=lax.Precision.HIGHEST)
        + rb_ref[...]
    )
    part = jnp.sum(r, axis=0, keepdims=True)

    @pl.when(i == 0)
    def _():
        acc_ref[...] = part

    @pl.when(i > 0)
    def _():
        acc_ref[...] += part

    t = jnp.tanh(acc_ref[...])
    out_ref[...] = (
        jnp.dot(t, l0w_ref[...], preferred_element_type=jnp.float32, precision=lax.Precision.HIGHEST)
        + l0b_ref[...]
    )


def kernel(x, edge_index, edge_attr, params):
    f32 = jnp.float32
    ei = edge_index.astype(jnp.int32)

    vw = jnp.pad(params["V_w"], ((0, 0), (0, 0), (0, H1P - H1)))
    vb = jnp.pad(params["V_b"], ((0, 0), (0, H1P - H1)))[:, None, :]
    ua = params["U_w"][:, :DF, :]
    ub = jnp.pad(params["U_w"][:, DF:DF + H1, :],
                 ((0, 0), (0, H1P - H1), (0, 0)))
    uc = params["U_w"][:, DF + H1:, :]
    u_bias = params["U_b"][:, None, :]
    ew = params["E_w"]
    eb = params["E_b"][None, :]
    rw = jnp.pad(params["R_w"], ((0, 0), (0, DF - H2)))
    rh, rx = rw[:DF], rw[DF:]
    rb = jnp.pad(params["R_b"], (0, DF - H2))[None, :]
    l0w = jnp.pad(params["lin0_w"], ((0, DF - H2), (0, 0)))
    l0w = jnp.pad(l0w, ((0, 0), (0, DF - 1)))
    l0b = jnp.pad(params["lin0_b"], (0, DF - 1))[None, :]

    zw = jnp.zeros((N, H1P), f32)
    ze = jnp.zeros((N, EH), f32)
    ones16 = jnp.ones((CH, EH), f32)

    hv = pl.pallas_call(
        _hv0_body,
        grid=(_GRID,),
        in_specs=[
            pl.BlockSpec((_BN, DF), lambda i: (i, 0)),
            _full((DF, H1P)),
            _full((1, H1P)),
        ],
        out_specs=pl.BlockSpec((_BN, H1P), lambda i: (i, 0)),
        out_shape=jax.ShapeDtypeStruct((N, H1P), f32),
    )(x, vw[0], vb[0])

    aggw, se, dg = _make_seg_layer0()(hv, edge_attr, ei, zw, ze, ones16)
    seg_gather = _make_seg_gather(H1P)

    layer_call = pl.pallas_call(
        _layer_body,
        grid=(_GRID,),
        in_specs=[
            pl.BlockSpec((_BN, DF), lambda i: (i, 0)),
            pl.BlockSpec((NC, _BN, H1P), lambda i: (0, i, 0)),
            pl.BlockSpec((NC, _BN, EH), lambda i: (0, i, 0)),
            pl.BlockSpec((NC, _BN, EH), lambda i: (0, i, 0)),
            _full((EH, EH)),
            _full((1, EH)),
            _full((DF, DF)),
            _full((H1P, DF)),
            _full((EH, DF)),
            _full((1, DF)),
            _full((DF, H1P)),
            _full((1, H1P)),
        ],
        out_specs=[
            pl.BlockSpec((_BN, DF), lambda i: (i, 0)),
            pl.BlockSpec((_BN, H1P), lambda i: (i, 0)),
        ],
        out_shape=[
            jax.ShapeDtypeStruct((N, DF), f32),
            jax.ShapeDtypeStruct((N, H1P), f32),
        ],
    )

    h = x
    for k in range(NLAYER - 1):
        if k > 0:
            aggw = seg_gather(hv, ei, zw)
        h, hv = layer_call(h, aggw, se, dg, ew, eb,
                           ua[k], ub[k], uc[k], u_bias[k],
                           vw[k + 1], vb[k + 1])

    aggw = seg_gather(hv, ei, zw)

    _, out = pl.pallas_call(
        _final_body,
        grid=(_GRID,),
        in_specs=[
            pl.BlockSpec((_BN, DF), lambda i: (i, 0)),
            pl.BlockSpec((_BN, DF), lambda i: (i, 0)),
            pl.BlockSpec((NC, _BN, H1P), lambda i: (0, i, 0)),
            pl.BlockSpec((NC, _BN, EH), lambda i: (0, i, 0)),
            pl.BlockSpec((NC, _BN, EH), lambda i: (0, i, 0)),
            _full((EH, EH)),
            _full((1, EH)),
            _full((DF, DF)),
            _full((H1P, DF)),
            _full((EH, DF)),
            _full((1, DF)),
            _full((DF, DF)),
            _full((DF, DF)),
            _full((1, DF)),
            _full((DF, DF)),
            _full((1, DF)),
        ],
        out_specs=[
            pl.BlockSpec((1, DF), lambda i: (0, 0)),
            pl.BlockSpec((1, DF), lambda i: (0, 0)),
        ],
        out_shape=[
            jax.ShapeDtypeStruct((1, DF), f32),
            jax.ShapeDtypeStruct((1, DF), f32),
        ],
    )(h, x, aggw, se, dg, ew, eb, ua[2], ub[2], uc[2], u_bias[2],
      rh, rx, rb, l0w, l0b)

    return out[:, :1]


# speedup vs baseline: 1.0661x; 1.0487x over previous
import functools

import jax
import jax.numpy as jnp
from jax import lax
from jax.experimental import pallas as pl
from jax.experimental.pallas import tpu as pltpu
from jax.experimental.pallas import tpu_sc as plsc

N = 10000
E = 320000
DF = 128
DE = 16
H1 = 50
H1P = 64
EH = 16
H2 = 80
NLAYER = 3

NC, NS = 2, 16
NW = NC * NS
CH = 80
CPW = E // NW // CH
RPT = N // NS

_SELU_ALPHA = 1.6732632423543772
_SELU_SCALE = 1.0507009873554805


def _selu(v):
    return _SELU_SCALE * jnp.where(v > 0.0, v, _SELU_ALPHA * (jnp.exp(v) - 1.0))


def _sc_mesh():
    return plsc.VectorSubcoreMesh(
        core_axis_name="c", subcore_axis_name="s", num_cores=NC, num_subcores=NS
    )


def _make_seg_gather(width):
    @functools.partial(
        pl.kernel,
        out_type=jax.ShapeDtypeStruct((NC, N, width), jnp.float32),
        mesh=_sc_mesh(),
        compiler_params=pltpu.CompilerParams(use_tc_tiling_on_sc=False),
        scratch_types=[
            pltpu.VMEM((CPW, CH), jnp.int32),
            pltpu.VMEM((CPW, CH), jnp.int32),
            pltpu.VMEM((3, CH, width), jnp.float32),
            pltpu.VMEM_SHARED((N, width), jnp.float32),
            pltpu.SemaphoreType.DMA,
            pltpu.SemaphoreType.DMA,
            pltpu.SemaphoreType.DMA,
        ],
    )
    def seg_gather(rows_hbm, src_hbm, dst_hbm, z_hbm, out_hbm,
                   src_v, dst_v, buf_v, acc_sh, sem0, sem1, sem2):
        sems = (sem0, sem1, sem2)
        c = lax.axis_index("c")
        s = lax.axis_index("s")
        wid = c * NS + s
        pltpu.sync_copy(z_hbm.at[pl.ds(s * RPT, RPT)],
                        acc_sh.at[pl.ds(s * RPT, RPT)])
        pltpu.sync_copy(src_hbm.at[pl.ds(wid * CPW, CPW)], src_v)
        pltpu.sync_copy(dst_hbm.at[pl.ds(wid * CPW, CPW)], dst_v)
        plsc.subcore_barrier()

        def gather(j, b):
            pltpu.async_copy(rows_hbm.at[src_v.at[j]], buf_v.at[b], sems[b])

        def drain_and_scatter(j, b):
            pltpu.make_async_copy(rows_hbm.at[src_v.at[j]], buf_v.at[b],
                                  sems[b]).wait()
            pltpu.sync_copy(buf_v.at[b], acc_sh.at[dst_v.at[j]], add=True)

        gather(0, 0)
        gather(1, 1)

        def body(g, carry):
            for b in range(3):
                j = g * 3 + b
                gather(j + 2, (b + 2) % 3)
                drain_and_scatter(j, b)
            return carry

        lax.fori_loop(0, (CPW - 2) // 3, body, 0)
        drain_and_scatter(CPW - 2, (CPW - 2) % 3)
        drain_and_scatter(CPW - 1, (CPW - 1) % 3)
        plsc.subcore_barrier()
        pltpu.sync_copy(acc_sh.at[pl.ds(s * RPT, RPT)],
                        out_hbm.at[c, pl.ds(s * RPT, RPT)])

    return seg_gather


def _make_seg_linear(width):
    @functools.partial(
        pl.kernel,
        out_type=jax.ShapeDtypeStruct((NC, N, width), jnp.float32),
        mesh=_sc_mesh(),
        compiler_params=pltpu.CompilerParams(use_tc_tiling_on_sc=False),
        scratch_types=[
            pltpu.VMEM((CPW, CH), jnp.int32),
            pltpu.VMEM((3, CH, width), jnp.float32),
            pltpu.VMEM_SHARED((N, width), jnp.float32),
            pltpu.SemaphoreType.DMA,
            pltpu.SemaphoreType.DMA,
            pltpu.SemaphoreType.DMA,
        ],
    )
    def seg_linear(me_hbm, dst_hbm, z_hbm, out_hbm, dst_v, buf_v, acc_sh,
                   sem0, sem1, sem2):
        sems = (sem0, sem1, sem2)
        c = lax.axis_index("c")
        s = lax.axis_index("s")
        wid = c * NS + s
        pltpu.sync_copy(z_hbm.at[pl.ds(s * RPT, RPT)],
                        acc_sh.at[pl.ds(s * RPT, RPT)])
        pltpu.sync_copy(dst_hbm.at[pl.ds(wid * CPW, CPW)], dst_v)
        plsc.subcore_barrier()

        def load(j, b):
            pltpu.async_copy(me_hbm.at[wid * CPW + j], buf_v.at[b], sems[b])

        def drain_and_scatter(j, b):
            pltpu.make_async_copy(me_hbm.at[wid * CPW + j], buf_v.at[b],
                                  sems[b]).wait()
            pltpu.sync_copy(buf_v.at[b], acc_sh.at[dst_v.at[j]], add=True)

        load(0, 0)
        load(1, 1)

        def body(g, carry):
            for b in range(3):
                j = g * 3 + b
                load(j + 2, (b + 2) % 3)
                drain_and_scatter(j, b)
            return carry

        lax.fori_loop(0, (CPW - 2) // 3, body, 0)
        drain_and_scatter(CPW - 2, (CPW - 2) % 3)
        drain_and_scatter(CPW - 1, (CPW - 1) % 3)
        plsc.subcore_barrier()
        pltpu.sync_copy(acc_sh.at[pl.ds(s * RPT, RPT)],
                        out_hbm.at[c, pl.ds(s * RPT, RPT)])

    return seg_linear


_BN = 2000
_GRID = N // _BN

_full = lambda shape: pl.BlockSpec(shape, lambda i: tuple(0 for _ in shape))


def _me_body(ea_ref, w_ref, b_ref, out_ref):
    out_ref[...] = (
        jnp.dot(ea_ref[...], w_ref[...], preferred_element_type=jnp.float32,
                precision=lax.Precision.HIGHEST)
        + b_ref[...]
    )


def _hv0_body(x_ref, vw_ref, vb_ref, out_ref):
    out_ref[...] = (
        jnp.dot(x_ref[...], vw_ref[...], preferred_element_type=jnp.float32,
                precision=lax.Precision.HIGHEST)
        + vb_ref[...]
    )


def _layer_h(h_ref, aw_ref, ae_ref, ua_ref, ub_ref, uc_ref, bias_ref):
    aw = aw_ref[0] + aw_ref[1]
    ae = ae_ref[0] + ae_ref[1]
    z = (
        jnp.dot(h_ref[...], ua_ref[...], preferred_element_type=jnp.float32,
                precision=lax.Precision.HIGHEST)
        + jnp.dot(aw, ub_ref[...], preferred_element_type=jnp.float32,
                  precision=lax.Precision.HIGHEST)
        + jnp.dot(ae, uc_ref[...], preferred_element_type=jnp.float32,
                  precision=lax.Precision.HIGHEST)
        + bias_ref[...]
    )
    return _selu(z)


def _layer_body(h_ref, aw_ref, ae_ref, ua_ref, ub_ref, uc_ref, bias_ref,
                vw_ref, vb_ref, h_out, hv_out):
    hn = _layer_h(h_ref, aw_ref, ae_ref, ua_ref, ub_ref, uc_ref, bias_ref)
    h_out[...] = hn
    hv_out[...] = (
        jnp.dot(hn, vw_ref[...], preferred_element_type=jnp.float32,
                precision=lax.Precision.HIGHEST)
        + vb_ref[...]
    )


def _final_body(h_ref, x_ref, aw_ref, ae_ref, ua_ref, ub_ref, uc_ref,
                bias_ref, rh_ref, rx_ref, rb_ref, l0w_ref, l0b_ref,
                acc_ref, out_ref):
    i = pl.program_id(0)
    hn = _layer_h(h_ref, aw_ref, ae_ref, ua_ref, ub_ref, uc_ref, bias_ref)
    r = _selu(
        jnp.dot(hn, rh_ref[...], preferred_element_type=jnp.float32,
                precision=lax.Precision.HIGHEST)
        + jnp.dot(x_ref[...], rx_ref[...], preferred_element_type=jnp.float32,
                  precision=lax.Precision.HIGHEST)
        + rb_ref[...]
    )
    part = jnp.sum(r, axis=0, keepdims=True)

    @pl.when(i == 0)
    def _():
        acc_ref[...] = part

    @pl.when(i > 0)
    def _():
        acc_ref[...] += part

    t = jnp.tanh(acc_ref[...])
    out_ref[...] = (
        jnp.dot(t, l0w_ref[...], preferred_element_type=jnp.float32,
                precision=lax.Precision.HIGHEST)
        + l0b_ref[...]
    )


def kernel(x, edge_index, edge_attr, params):
    f32 = jnp.float32
    src = edge_index[0].astype(jnp.int32).reshape(NW * CPW, CH)
    dst = edge_index[1].astype(jnp.int32).reshape(NW * CPW, CH)

    vw = jnp.pad(params["V_w"], ((0, 0), (0, 0), (0, H1P - H1)))
    vb = jnp.pad(params["V_b"], ((0, 0), (0, H1P - H1)))[:, None, :]
    ua = params["U_w"][:, :DF, :]
    ub = jnp.pad(params["U_w"][:, DF:DF + H1, :],
                 ((0, 0), (0, H1P - H1), (0, 0)))
    uc = params["U_w"][:, DF + H1:, :]
    u_bias = params["U_b"][:, None, :]
    ew = params["E_w"]
    w128 = jnp.zeros((DF, DF), f32)
    for i in range(8):
        w128 = w128.at[16 * i:16 * i + 16, 16 * i:16 * i + 16].set(ew)
    b128 = jnp.tile(params["E_b"], 8)[None, :]
    rw = jnp.pad(params["R_w"], ((0, 0), (0, DF - H2)))
    rh, rx = rw[:DF], rw[DF:]
    rb = jnp.pad(params["R_b"], (0, DF - H2))[None, :]
    l0w = jnp.pad(params["lin0_w"], ((0, DF - H2), (0, 0)))
    l0w = jnp.pad(l0w, ((0, 0), (0, DF - 1)))
    l0b = jnp.pad(params["lin0_b"], (0, DF - 1))[None, :]

    zw = jnp.zeros((N, H1P), f32)
    ze = jnp.zeros((N, EH), f32)

    ea8 = edge_attr.reshape(E // 8, DF)
    me = pl.pallas_call(
        _me_body,
        grid=(E // 8 // 4000,),
        in_specs=[
            pl.BlockSpec((4000, DF), lambda i: (i, 0)),
            _full((DF, DF)),
            _full((1, DF)),
        ],
        out_specs=pl.BlockSpec((4000, DF), lambda i: (i, 0)),
        out_shape=jax.ShapeDtypeStruct((E // 8, DF), f32),
    )(ea8, w128, b128)
    me = me.reshape(NW * CPW, CH, EH)

    hv = pl.pallas_call(
        _hv0_body,
        grid=(_GRID,),
        in_specs=[
            pl.BlockSpec((_BN, DF), lambda i: (i, 0)),
            _full((DF, H1P)),
            _full((1, H1P)),
        ],
        out_specs=pl.BlockSpec((_BN, H1P), lambda i: (i, 0)),
        out_shape=jax.ShapeDtypeStruct((N, H1P), f32),
    )(x, vw[0], vb[0])

    agge = _make_seg_linear(EH)(me, dst, ze)
    seg_gather = _make_seg_gather(H1P)

    layer_call = pl.pallas_call(
        _layer_body,
        grid=(_GRID,),
        in_specs=[
            pl.BlockSpec((_BN, DF), lambda i: (i, 0)),
            pl.BlockSpec((NC, _BN, H1P), lambda i: (0, i, 0)),
            pl.BlockSpec((NC, _BN, EH), lambda i: (0, i, 0)),
            _full((DF, DF)),
            _full((H1P, DF)),
            _full((EH, DF)),
            _full((1, DF)),
            _full((DF, H1P)),
            _full((1, H1P)),
        ],
        out_specs=[
            pl.BlockSpec((_BN, DF), lambda i: (i, 0)),
            pl.BlockSpec((_BN, H1P), lambda i: (i, 0)),
        ],
        out_shape=[
            jax.ShapeDtypeStruct((N, DF), f32),
            jax.ShapeDtypeStruct((N, H1P), f32),
        ],
    )

    h = x
    for k in range(NLAYER - 1):
        aggw = seg_gather(hv, src, dst, zw)
        h, hv = layer_call(h, aggw, agge, ua[k], ub[k], uc[k], u_bias[k],
                           vw[k + 1], vb[k + 1])

    aggw = seg_gather(hv, src, dst, zw)

    _, out = pl.pallas_call(
        _final_body,
        grid=(_GRID,),
        in_specs=[
            pl.BlockSpec((_BN, DF), lambda i: (i, 0)),
            pl.BlockSpec((_BN, DF), lambda i: (i, 0)),
            pl.BlockSpec((NC, _BN, H1P), lambda i: (0, i, 0)),
            pl.BlockSpec((NC, _BN, EH), lambda i: (0, i, 0)),
            _full((DF, DF)),
            _full((H1P, DF)),
            _full((EH, DF)),
            _full((1, DF)),
            _full((DF, DF)),
            _full((DF, DF)),
            _full((1, DF)),
            _full((DF, DF)),
            _full((1, DF)),
        ],
        out_specs=[
            pl.BlockSpec((1, DF), lambda i: (0, 0)),
            pl.BlockSpec((1, DF), lambda i: (0, 0)),
        ],
        out_shape=[
            jax.ShapeDtypeStruct((1, DF), f32),
            jax.ShapeDtypeStruct((1, DF), f32),
        ],
    )(h, x, aggw, agge, ua[2], ub[2], uc[2], u_bias[2],
      rh, rx, rb, l0w, l0b)

    return out[:, :1]
